# baseline (speedup 1.0000x reference)
import jax
import jax.numpy as jnp
from jax.experimental import pallas as pl
from jax.experimental.pallas import tpu as pltpu


def _add_kernel(x_ref, pos_ref, o_ref):
    o_ref[...] = x_ref[...] + pos_ref[...]


def kernel(x, pos_table):
    batch, seq_len, d_model = x.shape
    bs = 2048
    while seq_len % bs:
        bs //= 2
    grid = (seq_len // bs, batch)
    return pl.pallas_call(
        _add_kernel,
        grid=grid,
        in_specs=[
            pl.BlockSpec((1, bs, d_model), lambda s, b: (b, s, 0)),
            pl.BlockSpec((bs, d_model), lambda s, b: (s, 0)),
        ],
        out_specs=pl.BlockSpec((1, bs, d_model), lambda s, b: (b, s, 0)),
        out_shape=jax.ShapeDtypeStruct(x.shape, x.dtype),
        compiler_params=pltpu.CompilerParams(vmem_limit_bytes=128 * 1024 * 1024),
    )(x, pos_table)


# speedup vs baseline: 1.0044x; 1.0044x over previous
import jax
import jax.numpy as jnp
from jax.experimental import pallas as pl
from jax.experimental.pallas import tpu as pltpu

_NB = 3
_CH = 1024


def _make_body(n_chunks, chunks_per_seq, ch, d_model, seq_len):
    def body(x_hbm, pos_hbm, o_hbm, posbuf, xbuf, obuf, pos_sems, in_sems,
             out_sems):
        def pos_piece(i):
            sl = pl.ds(i * ch, ch)
            return pltpu.make_async_copy(
                pos_hbm.at[sl], posbuf.at[sl], pos_sems.at[i])

        def x_in(c):
            return pltpu.make_async_copy(
                x_hbm.at[pl.ds(c * ch, ch)], xbuf.at[c % _NB],
                in_sems.at[c % _NB])

        def x_out(c):
            return pltpu.make_async_copy(
                obuf.at[c % _NB], o_hbm.at[pl.ds(c * ch, ch)],
                out_sems.at[c % _NB])

        for i in range(chunks_per_seq):
            pos_piece(i).start()
        for c in range(_NB):
            x_in(c).start()
        for c in range(n_chunks):
            slot = c % _NB
            x_in(c).wait()
            if c < chunks_per_seq:
                pos_piece(c).wait()
            if c >= _NB:
                x_out(c - _NB).wait()
            obuf[slot] = xbuf[slot] + posbuf[pl.ds((c % chunks_per_seq) * ch,
                                                   ch)]
            x_out(c).start()
            if c + _NB < n_chunks:
                x_in(c + _NB).start()
        for c in range(n_chunks - _NB, n_chunks):
            x_out(c).wait()

    return body


def kernel(x, pos_table):
    batch, seq_len, d_model = x.shape
    n_rows = batch * seq_len
    ch = _CH
    while seq_len % ch:
        ch //= 2
    n_chunks = n_rows // ch
    chunks_per_seq = seq_len // ch
    xf = x.reshape(n_rows, d_model)
    body = _make_body(n_chunks, chunks_per_seq, ch, d_model, seq_len)
    out = pl.pallas_call(
        body,
        in_specs=[
            pl.BlockSpec(memory_space=pl.ANY),
            pl.BlockSpec(memory_space=pl.ANY),
        ],
        out_specs=pl.BlockSpec(memory_space=pl.ANY),
        out_shape=jax.ShapeDtypeStruct((n_rows, d_model), x.dtype),
        scratch_shapes=[
            pltpu.VMEM((seq_len, d_model), jnp.float32),
            pltpu.VMEM((_NB, ch, d_model), jnp.float32),
            pltpu.VMEM((_NB, ch, d_model), jnp.float32),
            pltpu.SemaphoreType.DMA((chunks_per_seq,)),
            pltpu.SemaphoreType.DMA((_NB,)),
            pltpu.SemaphoreType.DMA((_NB,)),
        ],
        compiler_params=pltpu.CompilerParams(
            vmem_limit_bytes=60 * 1024 * 1024),
    )(xf, pos_table)
    return out.reshape(batch, seq_len, d_model)


# speedup vs baseline: 1.0047x; 1.0003x over previous
import jax
import jax.numpy as jnp
from jax.experimental import pallas as pl
from jax.experimental.pallas import tpu as pltpu

_NB = 6
_CH = 512


def _make_body(n_chunks, chunks_per_seq, ch, d_model, seq_len):
    def body(x_hbm, pos_hbm, o_hbm, posbuf, xbuf, obuf, pos_sems, in_sems,
             out_sems):
        def pos_piece(i):
            sl = pl.ds(i * ch, ch)
            return pltpu.make_async_copy(
                pos_hbm.at[sl], posbuf.at[sl], pos_sems.at[i])

        def x_in(c):
            return pltpu.make_async_copy(
                x_hbm.at[pl.ds(c * ch, ch)], xbuf.at[c % _NB],
                in_sems.at[c % _NB])

        def x_out(c):
            return pltpu.make_async_copy(
                obuf.at[c % _NB], o_hbm.at[pl.ds(c * ch, ch)],
                out_sems.at[c % _NB])

        for i in range(chunks_per_seq):
            pos_piece(i).start()
        for c in range(_NB):
            x_in(c).start()
        for c in range(n_chunks):
            slot = c % _NB
            x_in(c).wait()
            if c < chunks_per_seq:
                pos_piece(c).wait()
            if c >= _NB:
                x_out(c - _NB).wait()
            obuf[slot] = xbuf[slot] + posbuf[pl.ds((c % chunks_per_seq) * ch,
                                                   ch)]
            x_out(c).start()
            if c + _NB < n_chunks:
                x_in(c + _NB).start()
        for c in range(n_chunks - _NB, n_chunks):
            x_out(c).wait()

    return body


def kernel(x, pos_table):
    batch, seq_len, d_model = x.shape
    n_rows = batch * seq_len
    ch = _CH
    while seq_len % ch:
        ch //= 2
    n_chunks = n_rows // ch
    chunks_per_seq = seq_len // ch
    xf = x.reshape(n_rows, d_model)
    body = _make_body(n_chunks, chunks_per_seq, ch, d_model, seq_len)
    out = pl.pallas_call(
        body,
        in_specs=[
            pl.BlockSpec(memory_space=pl.ANY),
            pl.BlockSpec(memory_space=pl.ANY),
        ],
        out_specs=pl.BlockSpec(memory_space=pl.ANY),
        out_shape=jax.ShapeDtypeStruct((n_rows, d_model), x.dtype),
        scratch_shapes=[
            pltpu.VMEM((seq_len, d_model), jnp.float32),
            pltpu.VMEM((_NB, ch, d_model), jnp.float32),
            pltpu.VMEM((_NB, ch, d_model), jnp.float32),
            pltpu.SemaphoreType.DMA((chunks_per_seq,)),
            pltpu.SemaphoreType.DMA((_NB,)),
            pltpu.SemaphoreType.DMA((_NB,)),
        ],
        compiler_params=pltpu.CompilerParams(
            vmem_limit_bytes=60 * 1024 * 1024),
    )(xf, pos_table)
    return out.reshape(batch, seq_len, d_model)
